# baseline (speedup 1.0000x reference)
import functools

import jax
import jax.numpy as jnp
from jax import lax
from jax.experimental import pallas as pl
from jax.experimental.pallas import tpu as pltpu
from jax.experimental.pallas import tpu_sc as plsc

B, S, D = 1, 2048, 768
E, K, H = 8, 2, 3072
T = B * S

BBT = 128
NSLOT = T * K
NPAD = NSLOT + E * BBT
NBLK = NPAD // BBT
NBLK_PAD = 64

NWC = 16
NW = 32
CHK = NSLOT // NWC
ZCH = NPAD // NWC
GCH = NPAD // NW
GSUB = GCH // 2
TPW = T // NW
CT = 16

_SC_MESH = plsc.VectorSubcoreMesh(core_axis_name="c", subcore_axis_name="s")


def _gating_kernel(tok_ref, wg_ref, topi_ref, topw_ref):
    logits = jnp.dot(tok_ref[...], wg_ref[...],
                     preferred_element_type=jnp.float32)
    m = jnp.max(logits, axis=-1, keepdims=True)
    ex = jnp.exp(logits - m)
    probs = ex / jnp.sum(ex, axis=-1, keepdims=True)

    eidx = jax.lax.broadcasted_iota(jnp.int32, (T, E), 1)
    big = jnp.int32(E + 1)

    v1 = jnp.max(probs, axis=-1, keepdims=True)
    i1 = jnp.min(jnp.where(probs == v1, eidx, big), axis=-1, keepdims=True)
    probs2 = jnp.where(eidx == i1, -jnp.inf, probs)
    v2 = jnp.max(probs2, axis=-1, keepdims=True)
    i2 = jnp.min(jnp.where(probs2 == v2, eidx, big), axis=-1, keepdims=True)

    s = v1 + v2
    topi_ref[...] = jnp.concatenate([i1, i2], axis=1)
    topw_ref[...] = jnp.concatenate([v1 / s, v2 / s], axis=1)


@functools.partial(
    pl.kernel,
    out_type=(jax.ShapeDtypeStruct((NSLOT,), jnp.int32),
              jax.ShapeDtypeStruct((NPAD,), jnp.int32),
              jax.ShapeDtypeStruct((NBLK_PAD,), jnp.int32),
              jax.ShapeDtypeStruct((NBLK_PAD,), jnp.int32)),
    mesh=_SC_MESH,
    compiler_params=pltpu.CompilerParams(needs_layout_passes=False),
    scratch_types=[
        pltpu.VMEM((CHK,), jnp.int32),
        pltpu.VMEM((CHK // 2,), jnp.int32),
        pltpu.VMEM((CHK // 2,), jnp.int32),
        pltpu.VMEM((CHK // 2,), jnp.int32),
        pltpu.VMEM((CHK // 2,), jnp.int32),
        pltpu.VMEM((16,), jnp.int32),
        pltpu.VMEM_SHARED((NSLOT,), jnp.int32),
        pltpu.VMEM((NSLOT,), jnp.int32),
        pltpu.VMEM((GCH,), jnp.int32),
        pltpu.VMEM_SHARED((NWC * 16,), jnp.int32),
        pltpu.VMEM((NWC * 16,), jnp.int32),
        pltpu.VMEM((16,), jnp.int32),
        pltpu.VMEM((16,), jnp.int32),
        pltpu.VMEM((32,), jnp.int32),
        pltpu.VMEM((16,), jnp.int32),
        pltpu.VMEM((NBLK_PAD,), jnp.int32),
        pltpu.VMEM((NBLK_PAD,), jnp.int32),
        pltpu.VMEM((32,), jnp.int32),
        pltpu.SemaphoreType.DMA,
    ],
)
def _sc_dispatch(eids_hbm, sp_hbm, ridx_hbm, blke_hbm, blkv_hbm,
                 eid_v, slotA_v, slotB_v, tokA_v, tokB_v, cnt_v,
                 sp_sh, spall_v, lridx_v, cntg_sh, cntg_v, off_v, pre_v,
                 bnd_v, gend_v, blke_v, blkv_v, scan_v, sem):
    cid = lax.axis_index("c")
    wid = lax.axis_index("s")
    is0 = cid == 0
    lanes = lax.iota(jnp.int32, 16)

    def scan16(x):
        scan_v[pl.ds(16, 16)] = x
        for sh in (1, 2, 4, 8):
            g = plsc.load_gather(
                scan_v, [jnp.maximum(lanes - sh, 0) + 16])
            x = x + jnp.where(lanes >= sh, g, 0)
            scan_v[pl.ds(16, 16)] = x
        return x

    base = wid * CHK
    pltpu.sync_copy(eids_hbm.at[pl.ds(base, CHK)], eid_v)

    NV = CHK // 16
    ranks = [jnp.zeros((16,), jnp.int32) for _ in range(NV)]
    cnt = jnp.zeros((16,), jnp.int32)
    fifteen = jnp.full((16,), 31, jnp.int32)
    for e in range(E):
        carry = jnp.zeros((16,), jnp.int32)
        for v in range(NV):
            vec = eid_v[pl.ds(v * 16, 16)]
            m = jnp.where(vec == e, 1, 0)
            cs = scan16(m)
            r = (cs - m) + carry
            ranks[v] = ranks[v] + jnp.where(vec == e, r, 0)
            carry = carry + plsc.load_gather(scan_v, [fifteen])
        cnt = jnp.where(lanes == e, carry, cnt)
    cnt_v[...] = cnt
    pltpu.sync_copy(cnt_v, cntg_sh.at[pl.ds(wid * 16, 16)])
    plsc.subcore_barrier()

    pltpu.sync_copy(cntg_sh, cntg_v)
    tot = jnp.zeros((16,), jnp.int32)
    pre = jnp.zeros((16,), jnp.int32)
    for w2 in range(NWC):
        row = cntg_v[pl.ds(w2 * 16, 16)]
        tot = tot + row
        pre = pre + jnp.where(jnp.int32(w2) < wid, row, 0)
    padded = ((tot + (BBT - 1)) // BBT) * BBT
    incl = scan16(padded)
    off = incl - padded
    off_v[...] = off
    pre_v[...] = pre
    bnd_v[pl.ds(16, 16)] = incl
    gend_v[...] = off + tot

    half = CHK // 2
    for v in range(NV):
        vec = eid_v[pl.ds(v * 16, 16)]
        b0 = plsc.load_gather(off_v, [vec]) + plsc.load_gather(pre_v, [vec])
        slot = b0 + ranks[v]
        tokv = (jnp.full((16,), base + v * 16, jnp.int32) + lanes) // K
        if v < NV // 2:
            slotA_v[pl.ds(v * 16, 16)] = slot
            tokA_v[pl.ds(v * 16, 16)] = tokv
        else:
            slotB_v[pl.ds((v - NV // 2) * 16, 16)] = slot
            tokB_v[pl.ds((v - NV // 2) * 16, 16)] = tokv

    pltpu.sync_copy(slotA_v, sp_sh.at[pl.ds(base, half)])
    pltpu.sync_copy(slotB_v, sp_sh.at[pl.ds(base + half, half)])
    plsc.subcore_barrier()
    pltpu.sync_copy(sp_sh, spall_v)

    gwid = wid * 2 + cid
    lo = gwid * GCH
    for v in range(GCH // 16):
        lridx_v[pl.ds(v * 16, 16)] = jnp.zeros((16,), jnp.int32)
    for v in range(NSLOT // 16):
        sl16 = spall_v[pl.ds(v * 16, 16)]
        rel = sl16 - lo
        msk = jnp.logical_and(rel >= 0, rel < GCH)
        tokv = (jnp.full((16,), v * 16, jnp.int32) + lanes) // K
        plsc.store_scatter(lridx_v, [jnp.clip(rel, 0, GCH - 1)], tokv,
                           mask=msk)
    pltpu.sync_copy(lridx_v, ridx_hbm.at[pl.ds(lo, GCH)])

    @pl.when(is0)
    def _():
        pltpu.sync_copy(slotA_v, sp_hbm.at[pl.ds(base, half)])
        pltpu.sync_copy(slotB_v, sp_hbm.at[pl.ds(base + half, half)])

    for vi in range(NBLK_PAD // 16):
        bstart = (lax.iota(jnp.int32, 16) + jnp.int32(vi * 16)) * BBT
        eidb = jnp.zeros((16,), jnp.int32)
        for e in range(E):
            be = plsc.load_gather(
                bnd_v, [jnp.full((16,), 16 + e, jnp.int32)])
            eidb = eidb + jnp.where(be <= bstart, 1, 0)
        eidb = jnp.minimum(eidb, E - 1)
        ge = plsc.load_gather(gend_v, [eidb])
        val = jnp.clip(ge - bstart, 0, BBT)
        blke_v[pl.ds(vi * 16, 16)] = eidb
        blkv_v[pl.ds(vi * 16, 16)] = val

    @pl.when(jnp.logical_and(is0, wid == 0))
    def _():
        pltpu.sync_copy(blke_v, blke_hbm)
        pltpu.sync_copy(blkv_v, blkv_hbm)


@functools.partial(
    pl.kernel,
    out_type=jax.ShapeDtypeStruct((NPAD, D), jnp.float32),
    mesh=_SC_MESH,
    compiler_params=pltpu.CompilerParams(needs_layout_passes=False),
    scratch_types=[
        pltpu.VMEM((GSUB,), jnp.int32),
        pltpu.VMEM((GSUB, D), jnp.float32),
        pltpu.SemaphoreType.DMA,
    ],
)
def _sc_gather(ridx_hbm, tok_hbm, xg_hbm, idx_v, rows_v, sem):
    wid = lax.axis_index("s") * 2 + lax.axis_index("c")
    for c in range(GCH // GSUB):
        base = wid * GCH + c * GSUB
        pltpu.sync_copy(ridx_hbm.at[pl.ds(base, GSUB)], idx_v)
        pltpu.sync_copy(tok_hbm.at[idx_v], rows_v)
        pltpu.sync_copy(rows_v, xg_hbm.at[pl.ds(base, GSUB)])


@functools.partial(
    pl.kernel,
    out_type=jax.ShapeDtypeStruct((T * D,), jnp.float32),
    mesh=_SC_MESH,
    compiler_params=pltpu.CompilerParams(needs_layout_passes=False),
    scratch_types=[
        pltpu.VMEM((2 * CT,), jnp.int32),
        pltpu.VMEM((16 + 2 * CT,), jnp.float32),
        pltpu.VMEM((2 * CT, D), jnp.float32),
        pltpu.VMEM((CT * D,), jnp.float32),
        pltpu.SemaphoreType.DMA,
    ],
)
def _sc_combine(yg_hbm, sp_hbm, tw_hbm, y_hbm, idx_v, w_v, rows_v,
                acc_v, sem):
    wid = lax.axis_index("s") * 2 + lax.axis_index("c")
    lanes = lax.iota(jnp.int32, 16)

    def body(c, _):
        sbase = wid * (2 * TPW) + c * (2 * CT)
        tbase = wid * TPW + c * CT
        pltpu.sync_copy(sp_hbm.at[pl.ds(sbase, 2 * CT)], idx_v)
        pltpu.sync_copy(tw_hbm.at[pl.ds(sbase, 2 * CT)],
                        w_v.at[pl.ds(16, 2 * CT)])
        pltpu.sync_copy(yg_hbm.at[idx_v], rows_v)
        for j in range(CT):
            r0 = jnp.full((16,), 2 * j, jnp.int32)
            r1 = jnp.full((16,), 2 * j + 1, jnp.int32)
            w0 = plsc.load_gather(w_v, [jnp.full((16,), 16 + 2 * j, jnp.int32)])
            w1 = plsc.load_gather(w_v, [jnp.full((16,), 17 + 2 * j, jnp.int32)])
            for d in range(D // 16):
                col = jnp.full((16,), d * 16, jnp.int32) + lanes
                a = plsc.load_gather(rows_v, [r0, col])
                b = plsc.load_gather(rows_v, [r1, col])
                acc_v[pl.ds(j * D + d * 16, 16)] = a * w0 + b * w1
        pltpu.sync_copy(acc_v, y_hbm.at[pl.ds(tbase * D, CT * D)])
        return 0

    lax.fori_loop(0, TPW // CT, body, 0)


def _ffn_grouped_kernel(eid_ref, valid_ref, xg_ref, w1_ref, b1_ref,
                        w2_ref, b2_ref, yg_ref):
    b = pl.program_id(0)

    @pl.when(valid_ref[b] > 0)
    def _():
        h = jnp.dot(xg_ref[...], w1_ref[0],
                    preferred_element_type=jnp.float32)
        h = h + b1_ref[0]
        a = jax.nn.gelu(h)
        o = jnp.dot(a, w2_ref[0], preferred_element_type=jnp.float32)
        yg_ref[...] = o + b2_ref[0]

    @pl.when(valid_ref[b] <= 0)
    def _():
        yg_ref[...] = jnp.zeros_like(yg_ref)


_USE_SC_DISPATCH = True
_USE_SC_GATHER = True


def _dispatch_host(topi):
    eids = topi.reshape(NSLOT)
    counts = jnp.bincount(eids, length=E).astype(jnp.int32)
    padded = ((counts + BBT - 1) // BBT) * BBT
    off = jnp.concatenate([jnp.zeros(1, jnp.int32),
                           jnp.cumsum(padded)[:-1].astype(jnp.int32)])
    gstart = jnp.concatenate([jnp.zeros(1, jnp.int32),
                              jnp.cumsum(counts)[:-1].astype(jnp.int32)])
    order = jnp.argsort(eids, stable=True)
    g = eids[order]
    slot_sorted = off[g] + (jnp.arange(NSLOT, dtype=jnp.int32) - gstart[g])
    slot = jnp.zeros(NSLOT, jnp.int32).at[order].set(slot_sorted)
    ridx = jnp.zeros(NPAD, jnp.int32).at[slot_sorted].set(
        (order // K).astype(jnp.int32))
    bounds = jnp.cumsum(padded).astype(jnp.int32)
    bstart = jnp.arange(NBLK_PAD, dtype=jnp.int32) * BBT
    blk_eid = jnp.minimum(
        jnp.sum(bounds[None, :] <= bstart[:, None], axis=1), E - 1
    ).astype(jnp.int32)
    gend_valid = off + counts
    blk_valid = jnp.clip(gend_valid[blk_eid] - bstart, 0, BBT)
    return slot, ridx, blk_eid, blk_valid


def kernel(x, Wg, W1, b1, W2, b2):
    tok = x.reshape(T, D)

    topi, topw = pl.pallas_call(
        _gating_kernel,
        out_shape=(jax.ShapeDtypeStruct((T, K), jnp.int32),
                   jax.ShapeDtypeStruct((T, K), jnp.float32)),
    )(tok, Wg)

    if _USE_SC_DISPATCH:
        sp, ridx, blk_eid, blk_valid = _sc_dispatch(topi.reshape(NSLOT))
    else:
        sp, ridx, blk_eid, blk_valid = _dispatch_host(topi)
    if _USE_SC_GATHER:
        xg = _sc_gather(ridx, tok)
    else:
        xg = tok[ridx]

    yg = pl.pallas_call(
        _ffn_grouped_kernel,
        grid_spec=pltpu.PrefetchScalarGridSpec(
            num_scalar_prefetch=2,
            grid=(NBLK,),
            in_specs=[
                pl.BlockSpec((BBT, D), lambda b, e_m, v_m: (b, 0)),
                pl.BlockSpec((1, D, H), lambda b, e_m, v_m: (e_m[b], 0, 0)),
                pl.BlockSpec((1, 1, H), lambda b, e_m, v_m: (e_m[b], 0, 0)),
                pl.BlockSpec((1, H, D), lambda b, e_m, v_m: (e_m[b], 0, 0)),
                pl.BlockSpec((1, 1, D), lambda b, e_m, v_m: (e_m[b], 0, 0)),
            ],
            out_specs=pl.BlockSpec((BBT, D), lambda b, e_m, v_m: (b, 0)),
        ),
        out_shape=jax.ShapeDtypeStruct((NPAD, D), jnp.float32),
    )(blk_eid, blk_valid, xg, W1, b1.reshape(E, 1, H), W2,
      b2.reshape(E, 1, D))

    y = _sc_combine(yg, sp, topw.reshape(NSLOT))
    return y.reshape(B, S, D)


_ = _sc_dispatch, _sc_gather


# speedup vs baseline: 1.0128x; 1.0128x over previous
import functools

import jax
import jax.numpy as jnp
from jax import lax
from jax.experimental import pallas as pl
from jax.experimental.pallas import tpu as pltpu
from jax.experimental.pallas import tpu_sc as plsc

B, S, D = 1, 2048, 768
E, K, H = 8, 2, 3072
T = B * S

BBT = 128
NSLOT = T * K
NPAD = NSLOT + E * BBT
NBLK = NPAD // BBT
NBLK_PAD = 64

NWC = 16
NW = 32
CHK = NSLOT // NWC
ZCH = NPAD // NWC
GCH = NPAD // NW
GSUB = GCH // 2
TPW = T // NW
CT = 16

_SC_MESH = plsc.VectorSubcoreMesh(core_axis_name="c", subcore_axis_name="s")


def _gating_kernel(tok_ref, wg_ref, topi_ref, topw_ref):
    logits = jnp.dot(tok_ref[...], wg_ref[...],
                     preferred_element_type=jnp.float32)
    m = jnp.max(logits, axis=-1, keepdims=True)
    ex = jnp.exp(logits - m)
    probs = ex / jnp.sum(ex, axis=-1, keepdims=True)

    eidx = jax.lax.broadcasted_iota(jnp.int32, (T, E), 1)
    big = jnp.int32(E + 1)

    v1 = jnp.max(probs, axis=-1, keepdims=True)
    i1 = jnp.min(jnp.where(probs == v1, eidx, big), axis=-1, keepdims=True)
    probs2 = jnp.where(eidx == i1, -jnp.inf, probs)
    v2 = jnp.max(probs2, axis=-1, keepdims=True)
    i2 = jnp.min(jnp.where(probs2 == v2, eidx, big), axis=-1, keepdims=True)

    s = v1 + v2
    topi_ref[...] = jnp.concatenate([i1, i2], axis=1)
    topw_ref[...] = jnp.concatenate([v1 / s, v2 / s], axis=1)


@functools.partial(
    pl.kernel,
    out_type=(jax.ShapeDtypeStruct((NSLOT,), jnp.int32),
              jax.ShapeDtypeStruct((NPAD, D), jnp.float32),
              jax.ShapeDtypeStruct((NBLK_PAD,), jnp.int32),
              jax.ShapeDtypeStruct((NBLK_PAD,), jnp.int32)),
    mesh=_SC_MESH,
    compiler_params=pltpu.CompilerParams(needs_layout_passes=False),
    scratch_types=[
        pltpu.VMEM((CHK,), jnp.int32),
        pltpu.VMEM((CHK // 2,), jnp.int32),
        pltpu.VMEM((CHK // 2,), jnp.int32),
        pltpu.VMEM((CHK // 2,), jnp.int32),
        pltpu.VMEM((CHK // 2,), jnp.int32),
        pltpu.VMEM((16,), jnp.int32),
        pltpu.VMEM_SHARED((NSLOT,), jnp.int32),
        pltpu.VMEM((NSLOT,), jnp.int32),
        pltpu.VMEM((GCH,), jnp.int32),
        pltpu.VMEM((GSUB, D), jnp.float32),
        pltpu.VMEM_SHARED((NWC * 16,), jnp.int32),
        pltpu.VMEM((NWC * 16,), jnp.int32),
        pltpu.VMEM((16,), jnp.int32),
        pltpu.VMEM((16,), jnp.int32),
        pltpu.VMEM((32,), jnp.int32),
        pltpu.VMEM((16,), jnp.int32),
        pltpu.VMEM((NBLK_PAD,), jnp.int32),
        pltpu.VMEM((NBLK_PAD,), jnp.int32),
        pltpu.VMEM((32,), jnp.int32),
        pltpu.SemaphoreType.DMA,
    ],
)
def _sc_dispatch(eids_hbm, tok_hbm, sp_hbm, xg_hbm, blke_hbm, blkv_hbm,
                 eid_v, slotA_v, slotB_v, tokA_v, tokB_v, cnt_v,
                 sp_sh, spall_v, lridx_v, rows_v, cntg_sh, cntg_v, off_v,
                 pre_v, bnd_v, gend_v, blke_v, blkv_v, scan_v, sem):
    cid = lax.axis_index("c")
    wid = lax.axis_index("s")
    is0 = cid == 0
    lanes = lax.iota(jnp.int32, 16)

    def scan16(x):
        scan_v[pl.ds(16, 16)] = x
        for sh in (1, 2, 4, 8):
            g = plsc.load_gather(
                scan_v, [jnp.maximum(lanes - sh, 0) + 16])
            x = x + jnp.where(lanes >= sh, g, 0)
            scan_v[pl.ds(16, 16)] = x
        return x

    base = wid * CHK
    pltpu.sync_copy(eids_hbm.at[pl.ds(base, CHK)], eid_v)

    NV = CHK // 16
    ranks = [jnp.zeros((16,), jnp.int32) for _ in range(NV)]
    cnt = jnp.zeros((16,), jnp.int32)
    fifteen = jnp.full((16,), 31, jnp.int32)
    for e in range(E):
        carry = jnp.zeros((16,), jnp.int32)
        for v in range(NV):
            vec = eid_v[pl.ds(v * 16, 16)]
            m = jnp.where(vec == e, 1, 0)
            cs = scan16(m)
            r = (cs - m) + carry
            ranks[v] = ranks[v] + jnp.where(vec == e, r, 0)
            carry = carry + plsc.load_gather(scan_v, [fifteen])
        cnt = jnp.where(lanes == e, carry, cnt)
    cnt_v[...] = cnt
    pltpu.sync_copy(cnt_v, cntg_sh.at[pl.ds(wid * 16, 16)])
    plsc.subcore_barrier()

    pltpu.sync_copy(cntg_sh, cntg_v)
    tot = jnp.zeros((16,), jnp.int32)
    pre = jnp.zeros((16,), jnp.int32)
    for w2 in range(NWC):
        row = cntg_v[pl.ds(w2 * 16, 16)]
        tot = tot + row
        pre = pre + jnp.where(jnp.int32(w2) < wid, row, 0)
    padded = ((tot + (BBT - 1)) // BBT) * BBT
    incl = scan16(padded)
    off = incl - padded
    off_v[...] = off
    pre_v[...] = pre
    bnd_v[pl.ds(16, 16)] = incl
    gend_v[...] = off + tot

    half = CHK // 2
    for v in range(NV):
        vec = eid_v[pl.ds(v * 16, 16)]
        b0 = plsc.load_gather(off_v, [vec]) + plsc.load_gather(pre_v, [vec])
        slot = b0 + ranks[v]
        tokv = (jnp.full((16,), base + v * 16, jnp.int32) + lanes) // K
        if v < NV // 2:
            slotA_v[pl.ds(v * 16, 16)] = slot
            tokA_v[pl.ds(v * 16, 16)] = tokv
        else:
            slotB_v[pl.ds((v - NV // 2) * 16, 16)] = slot
            tokB_v[pl.ds((v - NV // 2) * 16, 16)] = tokv

    pltpu.sync_copy(slotA_v, sp_sh.at[pl.ds(base, half)])
    pltpu.sync_copy(slotB_v, sp_sh.at[pl.ds(base + half, half)])
    plsc.subcore_barrier()
    pltpu.sync_copy(sp_sh, spall_v)

    gwid = wid * 2 + cid
    lo = gwid * GCH
    for v in range(GCH // 16):
        lridx_v[pl.ds(v * 16, 16)] = jnp.zeros((16,), jnp.int32)
    for v in range(NSLOT // 16):
        sl16 = spall_v[pl.ds(v * 16, 16)]
        rel = sl16 - lo
        msk = jnp.logical_and(rel >= 0, rel < GCH)
        tokv = (jnp.full((16,), v * 16, jnp.int32) + lanes) // K
        plsc.store_scatter(lridx_v, [jnp.clip(rel, 0, GCH - 1)], tokv,
                           mask=msk)
    for c in range(GCH // GSUB):
        pltpu.sync_copy(tok_hbm.at[lridx_v.at[pl.ds(c * GSUB, GSUB)]],
                        rows_v)
        pltpu.sync_copy(rows_v, xg_hbm.at[pl.ds(lo + c * GSUB, GSUB)])

    @pl.when(is0)
    def _():
        pltpu.sync_copy(slotA_v, sp_hbm.at[pl.ds(base, half)])
        pltpu.sync_copy(slotB_v, sp_hbm.at[pl.ds(base + half, half)])

    for vi in range(NBLK_PAD // 16):
        bstart = (lax.iota(jnp.int32, 16) + jnp.int32(vi * 16)) * BBT
        eidb = jnp.zeros((16,), jnp.int32)
        for e in range(E):
            be = plsc.load_gather(
                bnd_v, [jnp.full((16,), 16 + e, jnp.int32)])
            eidb = eidb + jnp.where(be <= bstart, 1, 0)
        eidb = jnp.minimum(eidb, E - 1)
        ge = plsc.load_gather(gend_v, [eidb])
        val = jnp.clip(ge - bstart, 0, BBT)
        blke_v[pl.ds(vi * 16, 16)] = eidb
        blkv_v[pl.ds(vi * 16, 16)] = val

    @pl.when(jnp.logical_and(is0, wid == 0))
    def _():
        pltpu.sync_copy(blke_v, blke_hbm)
        pltpu.sync_copy(blkv_v, blkv_hbm)


@functools.partial(
    pl.kernel,
    out_type=jax.ShapeDtypeStruct((NPAD, D), jnp.float32),
    mesh=_SC_MESH,
    compiler_params=pltpu.CompilerParams(needs_layout_passes=False),
    scratch_types=[
        pltpu.VMEM((GSUB,), jnp.int32),
        pltpu.VMEM((GSUB, D), jnp.float32),
        pltpu.SemaphoreType.DMA,
    ],
)
def _sc_gather(ridx_hbm, tok_hbm, xg_hbm, idx_v, rows_v, sem):
    wid = lax.axis_index("s") * 2 + lax.axis_index("c")
    for c in range(GCH // GSUB):
        base = wid * GCH + c * GSUB
        pltpu.sync_copy(ridx_hbm.at[pl.ds(base, GSUB)], idx_v)
        pltpu.sync_copy(tok_hbm.at[idx_v], rows_v)
        pltpu.sync_copy(rows_v, xg_hbm.at[pl.ds(base, GSUB)])


@functools.partial(
    pl.kernel,
    out_type=jax.ShapeDtypeStruct((T * D,), jnp.float32),
    mesh=_SC_MESH,
    compiler_params=pltpu.CompilerParams(needs_layout_passes=False),
    scratch_types=[
        pltpu.VMEM((2 * CT,), jnp.int32),
        pltpu.VMEM((16 + 2 * CT,), jnp.float32),
        pltpu.VMEM((2 * CT, D), jnp.float32),
        pltpu.VMEM((CT * D,), jnp.float32),
        pltpu.SemaphoreType.DMA,
    ],
)
def _sc_combine(yg_hbm, sp_hbm, tw_hbm, y_hbm, idx_v, w_v, rows_v,
                acc_v, sem):
    wid = lax.axis_index("s") * 2 + lax.axis_index("c")
    lanes = lax.iota(jnp.int32, 16)

    def body(c, _):
        sbase = wid * (2 * TPW) + c * (2 * CT)
        tbase = wid * TPW + c * CT
        pltpu.sync_copy(sp_hbm.at[pl.ds(sbase, 2 * CT)], idx_v)
        pltpu.sync_copy(tw_hbm.at[pl.ds(sbase, 2 * CT)],
                        w_v.at[pl.ds(16, 2 * CT)])
        pltpu.sync_copy(yg_hbm.at[idx_v], rows_v)
        for j in range(CT):
            r0 = jnp.full((16,), 2 * j, jnp.int32)
            r1 = jnp.full((16,), 2 * j + 1, jnp.int32)
            w0 = plsc.load_gather(w_v, [jnp.full((16,), 16 + 2 * j, jnp.int32)])
            w1 = plsc.load_gather(w_v, [jnp.full((16,), 17 + 2 * j, jnp.int32)])
            for d in range(D // 16):
                col = jnp.full((16,), d * 16, jnp.int32) + lanes
                a = plsc.load_gather(rows_v, [r0, col])
                b = plsc.load_gather(rows_v, [r1, col])
                acc_v[pl.ds(j * D + d * 16, 16)] = a * w0 + b * w1
        pltpu.sync_copy(acc_v, y_hbm.at[pl.ds(tbase * D, CT * D)])
        return 0

    lax.fori_loop(0, TPW // CT, body, 0)


def _ffn_grouped_kernel(eid_ref, valid_ref, xg_ref, w1_ref, b1_ref,
                        w2_ref, b2_ref, yg_ref):
    b = pl.program_id(0)

    @pl.when(valid_ref[b] > 0)
    def _():
        h = jnp.dot(xg_ref[...], w1_ref[0],
                    preferred_element_type=jnp.float32)
        h = h + b1_ref[0]
        a = jax.nn.gelu(h)
        o = jnp.dot(a, w2_ref[0], preferred_element_type=jnp.float32)
        yg_ref[...] = o + b2_ref[0]

    @pl.when(valid_ref[b] <= 0)
    def _():
        yg_ref[...] = jnp.zeros_like(yg_ref)


_USE_SC_DISPATCH = True
_USE_SC_GATHER = True


def _dispatch_host(topi):
    eids = topi.reshape(NSLOT)
    counts = jnp.bincount(eids, length=E).astype(jnp.int32)
    padded = ((counts + BBT - 1) // BBT) * BBT
    off = jnp.concatenate([jnp.zeros(1, jnp.int32),
                           jnp.cumsum(padded)[:-1].astype(jnp.int32)])
    gstart = jnp.concatenate([jnp.zeros(1, jnp.int32),
                              jnp.cumsum(counts)[:-1].astype(jnp.int32)])
    order = jnp.argsort(eids, stable=True)
    g = eids[order]
    slot_sorted = off[g] + (jnp.arange(NSLOT, dtype=jnp.int32) - gstart[g])
    slot = jnp.zeros(NSLOT, jnp.int32).at[order].set(slot_sorted)
    ridx = jnp.zeros(NPAD, jnp.int32).at[slot_sorted].set(
        (order // K).astype(jnp.int32))
    bounds = jnp.cumsum(padded).astype(jnp.int32)
    bstart = jnp.arange(NBLK_PAD, dtype=jnp.int32) * BBT
    blk_eid = jnp.minimum(
        jnp.sum(bounds[None, :] <= bstart[:, None], axis=1), E - 1
    ).astype(jnp.int32)
    gend_valid = off + counts
    blk_valid = jnp.clip(gend_valid[blk_eid] - bstart, 0, BBT)
    return slot, ridx, blk_eid, blk_valid


def kernel(x, Wg, W1, b1, W2, b2):
    tok = x.reshape(T, D)

    topi, topw = pl.pallas_call(
        _gating_kernel,
        out_shape=(jax.ShapeDtypeStruct((T, K), jnp.int32),
                   jax.ShapeDtypeStruct((T, K), jnp.float32)),
    )(tok, Wg)

    sp, xg, blk_eid, blk_valid = _sc_dispatch(topi.reshape(NSLOT), tok)

    yg = pl.pallas_call(
        _ffn_grouped_kernel,
        grid_spec=pltpu.PrefetchScalarGridSpec(
            num_scalar_prefetch=2,
            grid=(NBLK,),
            in_specs=[
                pl.BlockSpec((BBT, D), lambda b, e_m, v_m: (b, 0)),
                pl.BlockSpec((1, D, H), lambda b, e_m, v_m: (e_m[b], 0, 0)),
                pl.BlockSpec((1, 1, H), lambda b, e_m, v_m: (e_m[b], 0, 0)),
                pl.BlockSpec((1, H, D), lambda b, e_m, v_m: (e_m[b], 0, 0)),
                pl.BlockSpec((1, 1, D), lambda b, e_m, v_m: (e_m[b], 0, 0)),
            ],
            out_specs=pl.BlockSpec((BBT, D), lambda b, e_m, v_m: (b, 0)),
        ),
        out_shape=jax.ShapeDtypeStruct((NPAD, D), jnp.float32),
    )(blk_eid, blk_valid, xg, W1, b1.reshape(E, 1, H), W2,
      b2.reshape(E, 1, D))

    y = _sc_combine(yg, sp, topw.reshape(NSLOT))
    return y.reshape(B, S, D)


_ = _sc_dispatch, _sc_gather
